# Initial kernel scaffold; baseline (speedup 1.0000x reference)
import jax
import jax.numpy as jnp
from jax.experimental import pallas as pl


def kernel(in_feat, edge_index, edge_weight, W1, b1, W2, b2, W3, b3, W4, b4):
    raise NotImplementedError("write your pallas kernel here")


# speedup vs baseline: 2.9383x; 2.9383x over previous
import functools
import math

import jax
import jax.numpy as jnp
import numpy as np
from jax import lax
from jax.experimental import pallas as pl
from jax.experimental.pallas import tpu as pltpu
from jax.experimental.pallas import tpu_sc as plsc

N_NODES = 10000
D = 128
NC = 2
NS = 16
NW = NC * NS
CH = 128
N_PAD = 10240
ROWS_PER_TILE = N_PAD // NS
ZR = 128
CB = 16


def _thetas(d=2):
    ts = []
    for i in range(d + 1):
        p = (np.poly1d([0.5, 0.0]) ** i) * (np.poly1d([-0.5, 1.0]) ** (d - i))
        beta = math.gamma(i + 1) * math.gamma(d + 1 - i) / math.gamma(d + 2)
        c = np.asarray(p.coeffs, dtype=np.float64) / beta
        ts.append([float(c[d - j]) for j in range(d + 1)])
    return ts

THETAS = _thetas(2)


def _mlp2_body(x_ref, w1_ref, b1_ref, w2_ref, b2_ref, o_ref):
    h = jnp.dot(x_ref[...], w1_ref[...], preferred_element_type=jnp.float32,
                precision=lax.Precision.HIGHEST)
    h = jnp.maximum(h + b1_ref[...], 0.0)
    h = jnp.dot(h, w2_ref[...], preferred_element_type=jnp.float32,
                precision=lax.Precision.HIGHEST)
    o_ref[...] = jnp.maximum(h + b2_ref[...], 0.0)


def _mlp2(x, w1, b1, w2, b2, rows_blk=1000):
    n = x.shape[0]
    grid = n // rows_blk
    return pl.pallas_call(
        _mlp2_body,
        grid=(grid,),
        in_specs=[
            pl.BlockSpec((rows_blk, D), lambda i: (i, 0)),
            pl.BlockSpec((D, D), lambda i: (0, 0)),
            pl.BlockSpec((1, D), lambda i: (0, 0)),
            pl.BlockSpec((D, D), lambda i: (0, 0)),
            pl.BlockSpec((1, D), lambda i: (0, 0)),
        ],
        out_specs=pl.BlockSpec((rows_blk, D), lambda i: (i, 0)),
        out_shape=jax.ShapeDtypeStruct((n, D), jnp.float32),
    )(x, w1, b1.reshape(1, D), w2, b2.reshape(1, D))


def _combine_body(h_ref, p0_ref, p1_ref, o_ref):
    o_ref[...] = h_ref[...] - p0_ref[0] - p1_ref[0]


def _combine(h, partials, rows_blk=1000):
    n = h.shape[0]
    grid = n // rows_blk
    return pl.pallas_call(
        _combine_body,
        grid=(grid,),
        in_specs=[
            pl.BlockSpec((rows_blk, D), lambda i: (i, 0)),
            pl.BlockSpec((1, rows_blk, D), lambda i: (0, i, 0)),
            pl.BlockSpec((1, rows_blk, D), lambda i: (1, i, 0)),
        ],
        out_specs=pl.BlockSpec((rows_blk, D), lambda i: (i, 0)),
        out_shape=jax.ShapeDtypeStruct((n, D), jnp.float32),
    )(h, partials, partials)


def _final_body(h_ref, f1_ref, p0_ref, p1_ref, u0_ref, u12_ref, u2_ref,
                b3_ref, w4_ref, b4_ref, o_ref):
    z = jnp.dot(h_ref[...], u0_ref[...], preferred_element_type=jnp.float32,
                precision=lax.Precision.HIGHEST)
    z += jnp.dot(f1_ref[...], u12_ref[...], preferred_element_type=jnp.float32,
                precision=lax.Precision.HIGHEST)
    y2 = p0_ref[0] + p1_ref[0]
    z -= jnp.dot(y2, u2_ref[...], preferred_element_type=jnp.float32,
                precision=lax.Precision.HIGHEST)
    z = jnp.maximum(z + b3_ref[...], 0.0)
    o_ref[...] = jnp.dot(z, w4_ref[...], preferred_element_type=jnp.float32,
                precision=lax.Precision.HIGHEST) + b4_ref[...]


def _final(h, f1, partials2, u0, u12, u2, b3, w4, b4, rows_blk=1000):
    n = h.shape[0]
    nc = w4.shape[1]
    grid = n // rows_blk
    return pl.pallas_call(
        _final_body,
        grid=(grid,),
        in_specs=[
            pl.BlockSpec((rows_blk, D), lambda i: (i, 0)),
            pl.BlockSpec((rows_blk, D), lambda i: (i, 0)),
            pl.BlockSpec((1, rows_blk, D), lambda i: (0, i, 0)),
            pl.BlockSpec((1, rows_blk, D), lambda i: (1, i, 0)),
            pl.BlockSpec((D, D), lambda i: (0, 0)),
            pl.BlockSpec((D, D), lambda i: (0, 0)),
            pl.BlockSpec((D, D), lambda i: (0, 0)),
            pl.BlockSpec((1, D), lambda i: (0, 0)),
            pl.BlockSpec((D, nc), lambda i: (0, 0)),
            pl.BlockSpec((1, nc), lambda i: (0, 0)),
        ],
        out_specs=pl.BlockSpec((rows_blk, nc), lambda i: (i, 0)),
        out_shape=jax.ShapeDtypeStruct((n, nc), jnp.float32),
    )(h, f1, partials2, partials2, u0, u12, u2, b3.reshape(1, D), w4,
      b4.reshape(1, nc))


def _spmm_sc(f, srcp, dstp, wp, n_chunks):
    mesh = plsc.VectorSubcoreMesh(core_axis_name="c", subcore_axis_name="s")

    @functools.partial(
        pl.kernel,
        out_type=jax.ShapeDtypeStruct((NC, N_PAD, D), jnp.float32),
        mesh=mesh,
        scratch_types=[
            pltpu.VMEM((CB, CH), jnp.int32),
            pltpu.VMEM((CB, CH), jnp.int32),
            pltpu.VMEM((CB, CH), jnp.float32),
            pltpu.VMEM((CH, D), jnp.float32),
            pltpu.VMEM_SHARED((N_PAD, D), jnp.float32),
            pltpu.SemaphoreType.DMA,
        ],
    )
    def spmm_kernel(f_hbm, src_hbm, dst_hbm, w_hbm, out_hbm,
                    src_v, dst_v, w_v, rows_v, acc, sem):
        cid = lax.axis_index("c")
        sid = lax.axis_index("s")
        wid = cid * NS + sid

        @pl.loop(0, CH)
        def _zr(i):
            @pl.loop(0, D, step=16)
            def _zc(j):
                rows_v[i, pl.ds(j, 16)] = jnp.zeros((16,), jnp.float32)

        for r in range(ROWS_PER_TILE // ZR):
            pltpu.sync_copy(rows_v, acc.at[pl.ds(sid * ROWS_PER_TILE + r * ZR, ZR)])
        plsc.subcore_barrier()

        @pl.loop(0, n_chunks // CB)
        def _outer(o):
            pltpu.sync_copy(src_hbm.at[wid, pl.ds(o * CB, CB)], src_v)
            pltpu.sync_copy(dst_hbm.at[wid, pl.ds(o * CB, CB)], dst_v)
            pltpu.sync_copy(w_hbm.at[wid, pl.ds(o * CB, CB)], w_v)

            @pl.loop(0, CB)
            def _chunk(j):
                pltpu.async_copy(f_hbm.at[src_v.at[j]], rows_v, sem).wait()

                @pl.loop(0, CH, step=16)
                def _grp(g):
                    wvec = w_v[j, pl.ds(g, 16)]
                    for i16 in range(16):
                        wb = jnp.full((16,), wvec[i16], jnp.float32)
                        for dblk in range(D // 16):
                            sl = pl.ds(dblk * 16, 16)
                            rows_v[g + i16, sl] = rows_v[g + i16, sl] * wb

                pltpu.sync_copy(rows_v, acc.at[dst_v.at[j]], add=True)

        plsc.subcore_barrier()

        pltpu.sync_copy(
            acc.at[pl.ds(sid * ROWS_PER_TILE, ROWS_PER_TILE)],
            out_hbm.at[cid, pl.ds(sid * ROWS_PER_TILE, ROWS_PER_TILE)],
        )

    return spmm_kernel(f, srcp, dstp, wp)


def kernel(in_feat, edge_index, edge_weight, W1, b1, W2, b2, W3, b3, W4, b4):
    n = in_feat.shape[0]
    e = edge_index.shape[1]
    n_chunks = -(-e // (NW * CH * CB)) * CB
    e_pad = NW * n_chunks * CH
    pad = e_pad - e

    src = jnp.concatenate([edge_index[0], jnp.zeros((pad,), jnp.int32)])
    dst = jnp.concatenate([edge_index[1], jnp.zeros((pad,), jnp.int32)])
    w = jnp.concatenate([edge_weight, jnp.zeros((pad,), jnp.float32)])
    srcp = src.reshape(NW, n_chunks, CH)
    dstp = dst.reshape(NW, n_chunks, CH)
    wp = w.reshape(NW, n_chunks, CH)

    h = _mlp2(in_feat, W1, b1, W2, b2)
    p1 = _spmm_sc(h, srcp, dstp, wp, n_chunks)
    f1 = _combine(h, p1)
    p2 = _spmm_sc(f1, srcp, dstp, wp, n_chunks)

    t = THETAS
    w3b = [W3[i * D:(i + 1) * D] for i in range(3)]
    u0 = t[0][0] * w3b[0] + t[1][0] * w3b[1] + t[2][0] * w3b[2]
    u1 = t[0][1] * w3b[0] + t[1][1] * w3b[1] + t[2][1] * w3b[2]
    u2 = t[0][2] * w3b[0] + t[1][2] * w3b[1] + t[2][2] * w3b[2]

    return _final(h, f1, p2, u0, u1 + u2, u2, b3, W4, b4)


# speedup vs baseline: 3.3599x; 1.1435x over previous
import functools
import math

import jax
import jax.numpy as jnp
import numpy as np
from jax import lax
from jax.experimental import pallas as pl
from jax.experimental.pallas import tpu as pltpu
from jax.experimental.pallas import tpu_sc as plsc

N_NODES = 10000
D = 128
NC = 2
NS = 16
NW = NC * NS
CH = 128
N_PAD = 10240
ROWS_PER_TILE = N_PAD // NS
ZR = 128
CB = 16


def _thetas(d=2):
    ts = []
    for i in range(d + 1):
        p = (np.poly1d([0.5, 0.0]) ** i) * (np.poly1d([-0.5, 1.0]) ** (d - i))
        beta = math.gamma(i + 1) * math.gamma(d + 1 - i) / math.gamma(d + 2)
        c = np.asarray(p.coeffs, dtype=np.float64) / beta
        ts.append([float(c[d - j]) for j in range(d + 1)])
    return ts

THETAS = _thetas(2)


def _mlp2_body(x_ref, w1_ref, b1_ref, w2_ref, b2_ref, o_ref):
    h = jnp.dot(x_ref[...], w1_ref[...], preferred_element_type=jnp.float32,
                precision=lax.Precision.HIGHEST)
    h = jnp.maximum(h + b1_ref[...], 0.0)
    h = jnp.dot(h, w2_ref[...], preferred_element_type=jnp.float32,
                precision=lax.Precision.HIGHEST)
    o_ref[...] = jnp.maximum(h + b2_ref[...], 0.0)


def _mlp2(x, w1, b1, w2, b2, rows_blk=1000):
    n = x.shape[0]
    grid = n // rows_blk
    return pl.pallas_call(
        _mlp2_body,
        grid=(grid,),
        in_specs=[
            pl.BlockSpec((rows_blk, D), lambda i: (i, 0)),
            pl.BlockSpec((D, D), lambda i: (0, 0)),
            pl.BlockSpec((1, D), lambda i: (0, 0)),
            pl.BlockSpec((D, D), lambda i: (0, 0)),
            pl.BlockSpec((1, D), lambda i: (0, 0)),
        ],
        out_specs=pl.BlockSpec((rows_blk, D), lambda i: (i, 0)),
        out_shape=jax.ShapeDtypeStruct((n, D), jnp.float32),
    )(x, w1, b1.reshape(1, D), w2, b2.reshape(1, D))


def _combine_body(h_ref, p0_ref, p1_ref, o_ref):
    o_ref[...] = h_ref[...] - p0_ref[0] - p1_ref[0]


def _combine(h, partials, rows_blk=1000):
    n = h.shape[0]
    grid = n // rows_blk
    return pl.pallas_call(
        _combine_body,
        grid=(grid,),
        in_specs=[
            pl.BlockSpec((rows_blk, D), lambda i: (i, 0)),
            pl.BlockSpec((1, rows_blk, D), lambda i: (0, i, 0)),
            pl.BlockSpec((1, rows_blk, D), lambda i: (1, i, 0)),
        ],
        out_specs=pl.BlockSpec((rows_blk, D), lambda i: (i, 0)),
        out_shape=jax.ShapeDtypeStruct((n, D), jnp.float32),
    )(h, partials, partials)


def _final_body(h_ref, f1_ref, p0_ref, p1_ref, u0_ref, u12_ref, u2_ref,
                b3_ref, w4_ref, b4_ref, o_ref):
    z = jnp.dot(h_ref[...], u0_ref[...], preferred_element_type=jnp.float32,
                precision=lax.Precision.HIGHEST)
    z += jnp.dot(f1_ref[...], u12_ref[...], preferred_element_type=jnp.float32,
                precision=lax.Precision.HIGHEST)
    y2 = p0_ref[0] + p1_ref[0]
    z -= jnp.dot(y2, u2_ref[...], preferred_element_type=jnp.float32,
                precision=lax.Precision.HIGHEST)
    z = jnp.maximum(z + b3_ref[...], 0.0)
    o_ref[...] = jnp.dot(z, w4_ref[...], preferred_element_type=jnp.float32,
                precision=lax.Precision.HIGHEST) + b4_ref[...]


def _final(h, f1, partials2, u0, u12, u2, b3, w4, b4, rows_blk=1000):
    n = h.shape[0]
    nc = w4.shape[1]
    grid = n // rows_blk
    return pl.pallas_call(
        _final_body,
        grid=(grid,),
        in_specs=[
            pl.BlockSpec((rows_blk, D), lambda i: (i, 0)),
            pl.BlockSpec((rows_blk, D), lambda i: (i, 0)),
            pl.BlockSpec((1, rows_blk, D), lambda i: (0, i, 0)),
            pl.BlockSpec((1, rows_blk, D), lambda i: (1, i, 0)),
            pl.BlockSpec((D, D), lambda i: (0, 0)),
            pl.BlockSpec((D, D), lambda i: (0, 0)),
            pl.BlockSpec((D, D), lambda i: (0, 0)),
            pl.BlockSpec((1, D), lambda i: (0, 0)),
            pl.BlockSpec((D, nc), lambda i: (0, 0)),
            pl.BlockSpec((1, nc), lambda i: (0, 0)),
        ],
        out_specs=pl.BlockSpec((rows_blk, nc), lambda i: (i, 0)),
        out_shape=jax.ShapeDtypeStruct((n, nc), jnp.float32),
    )(h, f1, partials2, partials2, u0, u12, u2, b3.reshape(1, D), w4,
      b4.reshape(1, nc))


def _spmm_sc(f, srcp, dstp, wp, n_chunks):
    mesh = plsc.VectorSubcoreMesh(core_axis_name="c", subcore_axis_name="s")
    n_slabs = n_chunks // CB

    @functools.partial(
        pl.kernel,
        out_type=jax.ShapeDtypeStruct((NC, N_PAD, D), jnp.float32),
        mesh=mesh,
        scratch_types=[
            pltpu.VMEM((CB, CH), jnp.int32),
            pltpu.VMEM((CB, CH), jnp.int32),
            pltpu.VMEM((CB, CH), jnp.float32),
            pltpu.VMEM((CH, D), jnp.float32),
            pltpu.VMEM((CH, D), jnp.float32),
            pltpu.VMEM_SHARED((N_PAD, D), jnp.float32),
            pltpu.SemaphoreType.DMA,
            pltpu.SemaphoreType.DMA,
            pltpu.SemaphoreType.DMA,
            pltpu.SemaphoreType.DMA,
        ],
    )
    def spmm_kernel(f_hbm, src_hbm, dst_hbm, w_hbm, out_hbm,
                    src_v, dst_v, w_v, r0, r1, acc, g0, g1, s0, s1):
        rows = [r0, r1]
        gsem = [g0, g1]
        ssem = [s0, s1]
        cid = lax.axis_index("c")
        sid = lax.axis_index("s")
        wid = cid * NS + sid

        def g_start(jj, b):
            pltpu.async_copy(f_hbm.at[src_v.at[jj]], rows[b], gsem[b])

        def g_wait(jj, b):
            pltpu.make_async_copy(f_hbm.at[src_v.at[jj]], rows[b],
                                  gsem[b]).wait()

        def s_start(jj, b):
            pltpu.async_copy(rows[b], acc.at[dst_v.at[jj]], ssem[b],
                             add=True)

        def s_wait(jj, b):
            pltpu.make_async_copy(rows[b], acc.at[dst_v.at[jj]],
                                  ssem[b]).wait()

        @pl.loop(0, CH)
        def _zr(i):
            @pl.loop(0, D, step=16)
            def _zc(j):
                r0[i, pl.ds(j, 16)] = jnp.zeros((16,), jnp.float32)

        for r in range(ROWS_PER_TILE // ZR):
            pltpu.sync_copy(r0,
                            acc.at[pl.ds(sid * ROWS_PER_TILE + r * ZR, ZR)])
        plsc.subcore_barrier()

        @pl.loop(0, n_slabs)
        def _slab(o):
            @pl.when(o > 0)
            def _():
                s_wait(CB - 2, 0)
                s_wait(CB - 1, 1)

            pltpu.sync_copy(src_hbm.at[wid, pl.ds(o * CB, CB)], src_v)
            pltpu.sync_copy(dst_hbm.at[wid, pl.ds(o * CB, CB)], dst_v)
            pltpu.sync_copy(w_hbm.at[wid, pl.ds(o * CB, CB)], w_v)

            g_start(0, 0)

            for base in range(0, CB, 2):
                for b in range(2):
                    jj = base + b
                    g_wait(jj, b)

                    nb = 1 - b
                    if jj + 1 < CB:
                        if jj > 0:
                            s_wait(jj - 1, nb)
                        g_start(jj + 1, nb)

                    @pl.loop(0, CH, step=16)
                    def _grp(g, b=b, jj=jj):
                        wvec = w_v[jj, pl.ds(g, 16)]
                        wbs = [jnp.full((16,), wvec[i], jnp.float32)
                               for i in range(16)]
                        for i16 in range(0, 16, 2):
                            for dblk in range(D // 16):
                                sl = pl.ds(dblk * 16, 16)
                                ra = rows[b][g + i16, sl]
                                rb = rows[b][g + i16 + 1, sl]
                                rows[b][g + i16, sl] = ra * wbs[i16]
                                rows[b][g + i16 + 1, sl] = rb * wbs[i16 + 1]

                    s_start(jj, b)

        s_wait(CB - 2, 0)
        s_wait(CB - 1, 1)
        plsc.subcore_barrier()

        pltpu.sync_copy(
            acc.at[pl.ds(sid * ROWS_PER_TILE, ROWS_PER_TILE)],
            out_hbm.at[cid, pl.ds(sid * ROWS_PER_TILE, ROWS_PER_TILE)],
        )

    return spmm_kernel(f, srcp, dstp, wp)


def kernel(in_feat, edge_index, edge_weight, W1, b1, W2, b2, W3, b3, W4, b4):
    n = in_feat.shape[0]
    e = edge_index.shape[1]
    n_chunks = -(-e // (NW * CH * CB)) * CB
    e_pad = NW * n_chunks * CH
    pad = e_pad - e

    src = jnp.concatenate([edge_index[0], jnp.zeros((pad,), jnp.int32)])
    dst = jnp.concatenate([edge_index[1], jnp.zeros((pad,), jnp.int32)])
    w = jnp.concatenate([edge_weight, jnp.zeros((pad,), jnp.float32)])
    srcp = src.reshape(NW, n_chunks, CH)
    dstp = dst.reshape(NW, n_chunks, CH)
    wp = w.reshape(NW, n_chunks, CH)

    h = _mlp2(in_feat, W1, b1, W2, b2)
    p1 = _spmm_sc(h, srcp, dstp, wp, n_chunks)
    f1 = _combine(h, p1)
    p2 = _spmm_sc(f1, srcp, dstp, wp, n_chunks)

    t = THETAS
    w3b = [W3[i * D:(i + 1) * D] for i in range(3)]
    u0 = t[0][0] * w3b[0] + t[1][0] * w3b[1] + t[2][0] * w3b[2]
    u1 = t[0][1] * w3b[0] + t[1][1] * w3b[1] + t[2][1] * w3b[2]
    u2 = t[0][2] * w3b[0] + t[1][2] * w3b[1] + t[2][2] * w3b[2]

    return _final(h, f1, p2, u0, u1 + u2, u2, b3, W4, b4)


# speedup vs baseline: 3.5368x; 1.0527x over previous
import functools
import math

import jax
import jax.numpy as jnp
import numpy as np
from jax import lax
from jax.experimental import pallas as pl
from jax.experimental.pallas import tpu as pltpu
from jax.experimental.pallas import tpu_sc as plsc

N_NODES = 10000
D = 128
NC = 2
NS = 16
NW = NC * NS
CH = 64
N_PAD = 10240
ROWS_PER_TILE = N_PAD // NS
CB = 16
NBUF = 4


def _thetas(d=2):
    ts = []
    for i in range(d + 1):
        p = (np.poly1d([0.5, 0.0]) ** i) * (np.poly1d([-0.5, 1.0]) ** (d - i))
        beta = math.gamma(i + 1) * math.gamma(d + 1 - i) / math.gamma(d + 2)
        c = np.asarray(p.coeffs, dtype=np.float64) / beta
        ts.append([float(c[d - j]) for j in range(d + 1)])
    return ts

THETAS = _thetas(2)


def _mlp2_body(x_ref, w1_ref, b1_ref, w2_ref, b2_ref, o_ref):
    h = jnp.dot(x_ref[...], w1_ref[...], preferred_element_type=jnp.float32,
                precision=lax.Precision.HIGHEST)
    h = jnp.maximum(h + b1_ref[...], 0.0)
    h = jnp.dot(h, w2_ref[...], preferred_element_type=jnp.float32,
                precision=lax.Precision.HIGHEST)
    o_ref[...] = jnp.maximum(h + b2_ref[...], 0.0)


def _mlp2(x, w1, b1, w2, b2, rows_blk=1000):
    n = x.shape[0]
    grid = n // rows_blk
    return pl.pallas_call(
        _mlp2_body,
        grid=(grid,),
        in_specs=[
            pl.BlockSpec((rows_blk, D), lambda i: (i, 0)),
            pl.BlockSpec((D, D), lambda i: (0, 0)),
            pl.BlockSpec((1, D), lambda i: (0, 0)),
            pl.BlockSpec((D, D), lambda i: (0, 0)),
            pl.BlockSpec((1, D), lambda i: (0, 0)),
        ],
        out_specs=pl.BlockSpec((rows_blk, D), lambda i: (i, 0)),
        out_shape=jax.ShapeDtypeStruct((n, D), jnp.float32),
    )(x, w1, b1.reshape(1, D), w2, b2.reshape(1, D))


def _combine_body(h_ref, p0_ref, p1_ref, o_ref):
    o_ref[...] = h_ref[...] - p0_ref[0] - p1_ref[0]


def _combine(h, partials, rows_blk=1000):
    n = h.shape[0]
    grid = n // rows_blk
    return pl.pallas_call(
        _combine_body,
        grid=(grid,),
        in_specs=[
            pl.BlockSpec((rows_blk, D), lambda i: (i, 0)),
            pl.BlockSpec((1, rows_blk, D), lambda i: (0, i, 0)),
            pl.BlockSpec((1, rows_blk, D), lambda i: (1, i, 0)),
        ],
        out_specs=pl.BlockSpec((rows_blk, D), lambda i: (i, 0)),
        out_shape=jax.ShapeDtypeStruct((n, D), jnp.float32),
    )(h, partials, partials)


def _final_body(h_ref, f1_ref, p0_ref, p1_ref, u0_ref, u12_ref, u2_ref,
                b3_ref, w4_ref, b4_ref, o_ref):
    z = jnp.dot(h_ref[...], u0_ref[...], preferred_element_type=jnp.float32,
                precision=lax.Precision.HIGHEST)
    z += jnp.dot(f1_ref[...], u12_ref[...], preferred_element_type=jnp.float32,
                precision=lax.Precision.HIGHEST)
    y2 = p0_ref[0] + p1_ref[0]
    z -= jnp.dot(y2, u2_ref[...], preferred_element_type=jnp.float32,
                precision=lax.Precision.HIGHEST)
    z = jnp.maximum(z + b3_ref[...], 0.0)
    o_ref[...] = jnp.dot(z, w4_ref[...], preferred_element_type=jnp.float32,
                precision=lax.Precision.HIGHEST) + b4_ref[...]


def _final(h, f1, partials2, u0, u12, u2, b3, w4, b4, rows_blk=1000):
    n = h.shape[0]
    nc = w4.shape[1]
    grid = n // rows_blk
    return pl.pallas_call(
        _final_body,
        grid=(grid,),
        in_specs=[
            pl.BlockSpec((rows_blk, D), lambda i: (i, 0)),
            pl.BlockSpec((rows_blk, D), lambda i: (i, 0)),
            pl.BlockSpec((1, rows_blk, D), lambda i: (0, i, 0)),
            pl.BlockSpec((1, rows_blk, D), lambda i: (1, i, 0)),
            pl.BlockSpec((D, D), lambda i: (0, 0)),
            pl.BlockSpec((D, D), lambda i: (0, 0)),
            pl.BlockSpec((D, D), lambda i: (0, 0)),
            pl.BlockSpec((1, D), lambda i: (0, 0)),
            pl.BlockSpec((D, nc), lambda i: (0, 0)),
            pl.BlockSpec((1, nc), lambda i: (0, 0)),
        ],
        out_specs=pl.BlockSpec((rows_blk, nc), lambda i: (i, 0)),
        out_shape=jax.ShapeDtypeStruct((n, nc), jnp.float32),
    )(h, f1, partials2, partials2, u0, u12, u2, b3.reshape(1, D), w4,
      b4.reshape(1, nc))


def _spmm_sc(f, srcp, dstp, wp, n_chunks):
    mesh = plsc.VectorSubcoreMesh(core_axis_name="c", subcore_axis_name="s")
    n_slabs = n_chunks // CB

    @functools.partial(
        pl.kernel,
        out_type=jax.ShapeDtypeStruct((NC, N_PAD, D), jnp.float32),
        mesh=mesh,
        scratch_types=[
            pltpu.VMEM((CB, CH), jnp.int32),
            pltpu.VMEM((CB, CH), jnp.int32),
            pltpu.VMEM((CB, CH), jnp.float32),
            pltpu.VMEM((CH, D), jnp.float32),
            pltpu.VMEM((CH, D), jnp.float32),
            pltpu.VMEM((CH, D), jnp.float32),
            pltpu.VMEM((CH, D), jnp.float32),
            pltpu.VMEM_SHARED((N_PAD, D), jnp.float32),
            pltpu.SemaphoreType.DMA,
            pltpu.SemaphoreType.DMA,
            pltpu.SemaphoreType.DMA,
            pltpu.SemaphoreType.DMA,
            pltpu.SemaphoreType.DMA,
            pltpu.SemaphoreType.DMA,
            pltpu.SemaphoreType.DMA,
            pltpu.SemaphoreType.DMA,
        ],
    )
    def spmm_kernel(f_hbm, src_hbm, dst_hbm, w_hbm, out_hbm,
                    src_v, dst_v, w_v, r0, r1, r2, r3, acc,
                    g0, g1, g2, g3, s0, s1, s2, s3):
        rows = [r0, r1, r2, r3]
        gsem = [g0, g1, g2, g3]
        ssem = [s0, s1, s2, s3]
        cid = lax.axis_index("c")
        sid = lax.axis_index("s")
        wid = cid * NS + sid

        def g_start(jj, b):
            pltpu.async_copy(f_hbm.at[src_v.at[jj]], rows[b], gsem[b])

        def g_wait(jj, b):
            pltpu.make_async_copy(f_hbm.at[src_v.at[jj]], rows[b],
                                  gsem[b]).wait()

        def s_start(jj, b):
            pltpu.async_copy(rows[b], acc.at[dst_v.at[jj]], ssem[b],
                             add=True)

        def s_wait(jj, b):
            pltpu.make_async_copy(rows[b], acc.at[dst_v.at[jj]],
                                  ssem[b]).wait()

        @pl.loop(0, CH)
        def _zr(i):
            @pl.loop(0, D, step=16)
            def _zc(j):
                r0[i, pl.ds(j, 16)] = jnp.zeros((16,), jnp.float32)

        for r in range(ROWS_PER_TILE // CH):
            pltpu.sync_copy(r0,
                            acc.at[pl.ds(sid * ROWS_PER_TILE + r * CH, CH)])
        plsc.subcore_barrier()

        @pl.loop(0, n_slabs)
        def _slab(o):
            pltpu.sync_copy(src_hbm.at[wid, pl.ds(o * CB, CB)], src_v)
            pltpu.sync_copy(dst_hbm.at[wid, pl.ds(o * CB, CB)], dst_v)
            pltpu.sync_copy(w_hbm.at[wid, pl.ds(o * CB, CB)], w_v)

            g_start(0, 0)
            g_start(1, 1)
            g_start(2, 2)

            for jj in range(CB):
                b = jj % NBUF
                g_wait(jj, b)

                if jj + 3 < CB:
                    nb = (jj + 3) % NBUF
                    if jj > 0:
                        s_wait(jj - 1, nb)
                    g_start(jj + 3, nb)

                @pl.loop(0, CH, step=16)
                def _grp(g, b=b, jj=jj):
                    wvec = w_v[jj, pl.ds(g, 16)]
                    wbs = [jnp.full((16,), wvec[i], jnp.float32)
                           for i in range(16)]
                    for i16 in range(0, 16, 2):
                        for dblk in range(D // 16):
                            sl = pl.ds(dblk * 16, 16)
                            ra = rows[b][g + i16, sl]
                            rb = rows[b][g + i16 + 1, sl]
                            rows[b][g + i16, sl] = ra * wbs[i16]
                            rows[b][g + i16 + 1, sl] = rb * wbs[i16 + 1]

                s_start(jj, b)

            for jj in range(CB - NBUF, CB):
                s_wait(jj, jj % NBUF)

        plsc.subcore_barrier()

        pltpu.sync_copy(
            acc.at[pl.ds(sid * ROWS_PER_TILE, ROWS_PER_TILE)],
            out_hbm.at[cid, pl.ds(sid * ROWS_PER_TILE, ROWS_PER_TILE)],
        )

    return spmm_kernel(f, srcp, dstp, wp)


def kernel(in_feat, edge_index, edge_weight, W1, b1, W2, b2, W3, b3, W4, b4):
    n = in_feat.shape[0]
    e = edge_index.shape[1]
    n_chunks = -(-e // (NW * CH * CB)) * CB
    e_pad = NW * n_chunks * CH
    pad = e_pad - e

    src = jnp.concatenate([edge_index[0], jnp.zeros((pad,), jnp.int32)])
    dst = jnp.concatenate([edge_index[1], jnp.zeros((pad,), jnp.int32)])
    w = jnp.concatenate([edge_weight, jnp.zeros((pad,), jnp.float32)])
    srcp = src.reshape(NW, n_chunks, CH)
    dstp = dst.reshape(NW, n_chunks, CH)
    wp = w.reshape(NW, n_chunks, CH)

    h = _mlp2(in_feat, W1, b1, W2, b2)
    p1 = _spmm_sc(h, srcp, dstp, wp, n_chunks)
    f1 = _combine(h, p1)
    p2 = _spmm_sc(f1, srcp, dstp, wp, n_chunks)

    t = THETAS
    w3b = [W3[i * D:(i + 1) * D] for i in range(3)]
    u0 = t[0][0] * w3b[0] + t[1][0] * w3b[1] + t[2][0] * w3b[2]
    u1 = t[0][1] * w3b[0] + t[1][1] * w3b[1] + t[2][1] * w3b[2]
    u2 = t[0][2] * w3b[0] + t[1][2] * w3b[1] + t[2][2] * w3b[2]

    return _final(h, f1, p2, u0, u1 + u2, u2, b3, W4, b4)


# speedup vs baseline: 3.5371x; 1.0001x over previous
import functools
import math

import jax
import jax.numpy as jnp
import numpy as np
from jax import lax
from jax.experimental import pallas as pl
from jax.experimental.pallas import tpu as pltpu
from jax.experimental.pallas import tpu_sc as plsc

N_NODES = 10000
D = 128
NC = 2
NS = 16
NW = NC * NS
CH = 64
N_PAD = 10240
ROWS_PER_TILE = N_PAD // NS
CB = 16
NBUF = 4


def _thetas(d=2):
    ts = []
    for i in range(d + 1):
        p = (np.poly1d([0.5, 0.0]) ** i) * (np.poly1d([-0.5, 1.0]) ** (d - i))
        beta = math.gamma(i + 1) * math.gamma(d + 1 - i) / math.gamma(d + 2)
        c = np.asarray(p.coeffs, dtype=np.float64) / beta
        ts.append([float(c[d - j]) for j in range(d + 1)])
    return ts

THETAS = _thetas(2)


def _mlp2_body(x_ref, w1_ref, b1_ref, w2_ref, b2_ref, o_ref):
    h = jnp.dot(x_ref[...], w1_ref[...], preferred_element_type=jnp.float32,
                precision=lax.Precision.HIGHEST)
    h = jnp.maximum(h + b1_ref[...], 0.0)
    h = jnp.dot(h, w2_ref[...], preferred_element_type=jnp.float32,
                precision=lax.Precision.HIGHEST)
    o_ref[...] = jnp.maximum(h + b2_ref[...], 0.0)


def _mlp2(x, w1, b1, w2, b2, rows_blk=1000):
    n = x.shape[0]
    grid = n // rows_blk
    return pl.pallas_call(
        _mlp2_body,
        grid=(grid,),
        in_specs=[
            pl.BlockSpec((rows_blk, D), lambda i: (i, 0)),
            pl.BlockSpec((D, D), lambda i: (0, 0)),
            pl.BlockSpec((1, D), lambda i: (0, 0)),
            pl.BlockSpec((D, D), lambda i: (0, 0)),
            pl.BlockSpec((1, D), lambda i: (0, 0)),
        ],
        out_specs=pl.BlockSpec((rows_blk, D), lambda i: (i, 0)),
        out_shape=jax.ShapeDtypeStruct((n, D), jnp.float32),
    )(x, w1, b1.reshape(1, D), w2, b2.reshape(1, D))


def _combine_body(h_ref, p0_ref, p1_ref, o_ref):
    o_ref[...] = h_ref[...] - p0_ref[0] - p1_ref[0]


def _combine(h, partials, rows_blk=1000):
    n = h.shape[0]
    grid = n // rows_blk
    return pl.pallas_call(
        _combine_body,
        grid=(grid,),
        in_specs=[
            pl.BlockSpec((rows_blk, D), lambda i: (i, 0)),
            pl.BlockSpec((1, rows_blk, D), lambda i: (0, i, 0)),
            pl.BlockSpec((1, rows_blk, D), lambda i: (1, i, 0)),
        ],
        out_specs=pl.BlockSpec((rows_blk, D), lambda i: (i, 0)),
        out_shape=jax.ShapeDtypeStruct((n, D), jnp.float32),
    )(h, partials, partials)


def _final_body(h_ref, f1_ref, p0_ref, p1_ref, u0_ref, u12_ref, u2_ref,
                b3_ref, w4_ref, b4_ref, o_ref):
    z = jnp.dot(h_ref[...], u0_ref[...], preferred_element_type=jnp.float32,
                precision=lax.Precision.HIGHEST)
    z += jnp.dot(f1_ref[...], u12_ref[...], preferred_element_type=jnp.float32,
                precision=lax.Precision.HIGHEST)
    y2 = p0_ref[0] + p1_ref[0]
    z -= jnp.dot(y2, u2_ref[...], preferred_element_type=jnp.float32,
                precision=lax.Precision.HIGHEST)
    z = jnp.maximum(z + b3_ref[...], 0.0)
    o_ref[...] = jnp.dot(z, w4_ref[...], preferred_element_type=jnp.float32,
                precision=lax.Precision.HIGHEST) + b4_ref[...]


def _final(h, f1, partials2, u0, u12, u2, b3, w4, b4, rows_blk=1000):
    n = h.shape[0]
    nc = w4.shape[1]
    grid = n // rows_blk
    return pl.pallas_call(
        _final_body,
        grid=(grid,),
        in_specs=[
            pl.BlockSpec((rows_blk, D), lambda i: (i, 0)),
            pl.BlockSpec((rows_blk, D), lambda i: (i, 0)),
            pl.BlockSpec((1, rows_blk, D), lambda i: (0, i, 0)),
            pl.BlockSpec((1, rows_blk, D), lambda i: (1, i, 0)),
            pl.BlockSpec((D, D), lambda i: (0, 0)),
            pl.BlockSpec((D, D), lambda i: (0, 0)),
            pl.BlockSpec((D, D), lambda i: (0, 0)),
            pl.BlockSpec((1, D), lambda i: (0, 0)),
            pl.BlockSpec((D, nc), lambda i: (0, 0)),
            pl.BlockSpec((1, nc), lambda i: (0, 0)),
        ],
        out_specs=pl.BlockSpec((rows_blk, nc), lambda i: (i, 0)),
        out_shape=jax.ShapeDtypeStruct((n, nc), jnp.float32),
    )(h, f1, partials2, partials2, u0, u12, u2, b3.reshape(1, D), w4,
      b4.reshape(1, nc))


def _spmm_sc(f, srcp, dstp, wp, n_chunks):
    mesh = plsc.VectorSubcoreMesh(core_axis_name="c", subcore_axis_name="s")
    n_slabs = n_chunks // CB

    @functools.partial(
        pl.kernel,
        out_type=jax.ShapeDtypeStruct((NC, N_PAD, D), jnp.float32),
        mesh=mesh,
        scratch_types=[
            pltpu.VMEM((CB, CH), jnp.int32),
            pltpu.VMEM((CB, CH), jnp.int32),
            pltpu.VMEM((CB, CH), jnp.float32),
            pltpu.VMEM((CH, D), jnp.float32),
            pltpu.VMEM((CH, D), jnp.float32),
            pltpu.VMEM((CH, D), jnp.float32),
            pltpu.VMEM((CH, D), jnp.float32),
            pltpu.VMEM_SHARED((N_PAD, D), jnp.float32),
            pltpu.SemaphoreType.DMA,
            pltpu.SemaphoreType.DMA,
            pltpu.SemaphoreType.DMA,
            pltpu.SemaphoreType.DMA,
            pltpu.SemaphoreType.DMA,
            pltpu.SemaphoreType.DMA,
            pltpu.SemaphoreType.DMA,
            pltpu.SemaphoreType.DMA,
        ],
    )
    def spmm_kernel(f_hbm, src_hbm, dst_hbm, w_hbm, out_hbm,
                    src_v, dst_v, w_v, r0, r1, r2, r3, acc,
                    g0, g1, g2, g3, s0, s1, s2, s3):
        rows = [r0, r1, r2, r3]
        gsem = [g0, g1, g2, g3]
        ssem = [s0, s1, s2, s3]
        cid = lax.axis_index("c")
        sid = lax.axis_index("s")
        wid = cid * NS + sid

        def g_start(jj, b):
            pltpu.async_copy(f_hbm.at[src_v.at[jj]], rows[b], gsem[b])

        def g_wait(jj, b):
            pltpu.make_async_copy(f_hbm.at[src_v.at[jj]], rows[b],
                                  gsem[b]).wait()

        def s_start(jj, b):
            pltpu.async_copy(rows[b], acc.at[dst_v.at[jj]], ssem[b],
                             add=True)

        def s_wait(jj, b):
            pltpu.make_async_copy(rows[b], acc.at[dst_v.at[jj]],
                                  ssem[b]).wait()

        @pl.loop(0, CH)
        def _zr(i):
            @pl.loop(0, D, step=16)
            def _zc(j):
                r0[i, pl.ds(j, 16)] = jnp.zeros((16,), jnp.float32)

        for r in range(ROWS_PER_TILE // CH):
            pltpu.sync_copy(r0,
                            acc.at[pl.ds(sid * ROWS_PER_TILE + r * CH, CH)])
        plsc.subcore_barrier()

        @pl.loop(0, n_slabs)
        def _slab(o):
            pltpu.sync_copy(src_hbm.at[wid, pl.ds(o * CB, CB)], src_v)
            pltpu.sync_copy(dst_hbm.at[wid, pl.ds(o * CB, CB)], dst_v)
            pltpu.sync_copy(w_hbm.at[wid, pl.ds(o * CB, CB)], w_v)

            g_start(0, 0)
            g_start(1, 1)
            g_start(2, 2)

            for jj in range(CB):
                b = jj % NBUF
                g_wait(jj, b)

                if jj + 3 < CB:
                    nb = (jj + 3) % NBUF
                    if jj > 0:
                        s_wait(jj - 1, nb)
                    g_start(jj + 3, nb)

                @pl.loop(0, CH, step=16)
                def _grp(g, b=b, jj=jj):
                    wvec = w_v[jj, pl.ds(g, 16)]
                    wbs = [jnp.full((16,), wvec[i], jnp.float32)
                           for i in range(16)]
                    for i16 in range(0, 16, 2):
                        for dblk in range(D // 16):
                            sl = pl.ds(dblk * 16, 16)
                            ra = rows[b][g + i16, sl]
                            rb = rows[b][g + i16 + 1, sl]
                            rows[b][g + i16, sl] = ra * wbs[i16]
                            rows[b][g + i16 + 1, sl] = rb * wbs[i16 + 1]

                s_start(jj, b)

            for jj in range(CB - NBUF, CB):
                s_wait(jj, jj % NBUF)

        plsc.subcore_barrier()

        pltpu.sync_copy(
            acc.at[pl.ds(sid * ROWS_PER_TILE, ROWS_PER_TILE)],
            out_hbm.at[cid, pl.ds(sid * ROWS_PER_TILE, ROWS_PER_TILE)],
        )

    return spmm_kernel(f, srcp, dstp, wp)


def kernel(in_feat, edge_index, edge_weight, W1, b1, W2, b2, W3, b3, W4, b4):
    n = in_feat.shape[0]
    e = edge_index.shape[1]
    n_chunks = -(-e // (NW * CH * CB)) * CB
    e_pad = NW * n_chunks * CH
    pad = e_pad - e

    pad_dst = n + jnp.arange(pad, dtype=jnp.int32) % (N_PAD - n)
    src = jnp.concatenate([edge_index[0], jnp.zeros((pad,), jnp.int32)])
    dst = jnp.concatenate([edge_index[1], pad_dst])
    w = jnp.concatenate([edge_weight, jnp.zeros((pad,), jnp.float32)])
    srcp = src.reshape(NW, n_chunks, CH)
    dstp = dst.reshape(NW, n_chunks, CH)
    wp = w.reshape(NW, n_chunks, CH)

    h = _mlp2(in_feat, W1, b1, W2, b2)
    p1 = _spmm_sc(h, srcp, dstp, wp, n_chunks)
    f1 = _combine(h, p1)
    p2 = _spmm_sc(f1, srcp, dstp, wp, n_chunks)

    t = THETAS
    w3b = [W3[i * D:(i + 1) * D] for i in range(3)]
    u0 = t[0][0] * w3b[0] + t[1][0] * w3b[1] + t[2][0] * w3b[2]
    u1 = t[0][1] * w3b[0] + t[1][1] * w3b[1] + t[2][1] * w3b[2]
    u2 = t[0][2] * w3b[0] + t[1][2] * w3b[1] + t[2][2] * w3b[2]

    return _final(h, f1, p2, u0, u1 + u2, u2, b3, W4, b4)
